# Initial kernel scaffold; baseline (speedup 1.0000x reference)
import jax
import jax.numpy as jnp
from jax.experimental import pallas as pl


def kernel(cent, idx, mask):
    raise NotImplementedError("write your pallas kernel here")


# speedup vs baseline: 233.9718x; 233.9718x over previous
import functools

import jax
import jax.numpy as jnp
from jax import lax
from jax.experimental import pallas as pl
from jax.experimental.pallas import tpu as pltpu
from jax.experimental.pallas import tpu_sc as plsc

_K = 8192
_SHAPE = (4096, 4096)
_N = _SHAPE[0] * _SHAPE[1]

_NC = 2
_NS = 16
_NW = _NC * _NS
_LANES = 16

_PER_W = _N // _NW
_CHUNK = 16384
_NCHUNKS = _PER_W // _CHUNK


def _sc_body(cent_hbm, idx_hbm, mask_hbm, out_hbm,
             cent_ref, idx_ref, mask_ref, out_ref):
    wid = lax.axis_index("s") * _NC + lax.axis_index("c")
    base_w = wid * _PER_W

    pltpu.sync_copy(cent_hbm, cent_ref)

    def chunk_body(chunk, carry):
        base = base_w + chunk * _CHUNK
        pltpu.sync_copy(idx_hbm.at[pl.ds(base, _CHUNK)], idx_ref)
        pltpu.sync_copy(mask_hbm.at[pl.ds(base, _CHUNK)], mask_ref)

        def inner(i, c):
            off = pl.multiple_of(i * _LANES, _LANES)
            iv = idx_ref[pl.ds(off, _LANES)]
            sv = jnp.maximum(iv, 0)
            g = plsc.load_gather(cent_ref, [sv])
            out_ref[pl.ds(off, _LANES)] = g * mask_ref[pl.ds(off, _LANES)]
            return c

        lax.fori_loop(0, _CHUNK // _LANES, inner, 0, unroll=4)
        pltpu.sync_copy(out_ref, out_hbm.at[pl.ds(base, _CHUNK)])
        return carry

    lax.fori_loop(0, _NCHUNKS, chunk_body, 0)


def kernel(cent, idx, mask):
    idx_flat = idx.reshape(_N)
    mask_flat = mask.reshape(_N)

    mesh = plsc.VectorSubcoreMesh(core_axis_name="c", subcore_axis_name="s")
    out = pl.kernel(
        _sc_body,
        mesh=mesh,
        compiler_params=pltpu.CompilerParams(needs_layout_passes=False),
        out_type=jax.ShapeDtypeStruct((_N,), jnp.float32),
        scratch_types=[
            pltpu.VMEM((_K,), jnp.float32),
            pltpu.VMEM((_CHUNK,), jnp.int32),
            pltpu.VMEM((_CHUNK,), jnp.float32),
            pltpu.VMEM((_CHUNK,), jnp.float32),
        ],
    )(cent, idx_flat, mask_flat)
    return out.reshape(_SHAPE)


# speedup vs baseline: 610.1461x; 2.6078x over previous
import functools

import jax
import jax.numpy as jnp
from jax import lax
from jax.experimental import pallas as pl
from jax.experimental.pallas import tpu as pltpu
from jax.experimental.pallas import tpu_sc as plsc

_K = 8192
_SHAPE = (4096, 4096)
_N = _SHAPE[0] * _SHAPE[1]

_NC = 2
_NS = 16
_NW = _NC * _NS
_LANES = 16

_PER_W = _N // _NW
_CHUNK = 16384
_NCHUNKS = _PER_W // _CHUNK
_NBUF = 2


def _sc_body(cent_hbm, idx_hbm, mask_hbm, out_hbm,
             cent_ref, idx0, idx1, mask0, mask1, out0, out1,
             sem_in0, sem_in1, sem_out0, sem_out1):
    idx_bufs = (idx0, idx1)
    mask_bufs = (mask0, mask1)
    out_bufs = (out0, out1)
    sem_in = (sem_in0, sem_in1)
    sem_out = (sem_out0, sem_out1)

    wid = lax.axis_index("s") * _NC + lax.axis_index("c")
    base_w = wid * _PER_W

    pltpu.sync_copy(cent_hbm, cent_ref)

    def start_in(c, b):
        base = base_w + c * _CHUNK
        pltpu.async_copy(idx_hbm.at[pl.ds(base, _CHUNK)], idx_bufs[b],
                         sem_in[b])
        pltpu.async_copy(mask_hbm.at[pl.ds(base, _CHUNK)], mask_bufs[b],
                         sem_in[b])

    for b in range(_NBUF):
        start_in(b, b)

    def outer(g, carry):
        for b in range(_NBUF):
            c = _NBUF * g + b
            base = base_w + c * _CHUNK
            ib, mb, ob = idx_bufs[b], mask_bufs[b], out_bufs[b]

            pltpu.make_async_copy(idx_hbm.at[pl.ds(base, _CHUNK)], ib,
                                  sem_in[b]).wait()
            pltpu.make_async_copy(mask_hbm.at[pl.ds(base, _CHUNK)], mb,
                                  sem_in[b]).wait()

            @pl.when(g > 0)
            def _():
                prev = base_w + (c - _NBUF) * _CHUNK
                pltpu.make_async_copy(ob, out_hbm.at[pl.ds(prev, _CHUNK)],
                                      sem_out[b]).wait()

            @plsc.parallel_loop(0, _CHUNK, _LANES, unroll=8)
            def _(off):
                iv = ib[pl.ds(off, _LANES)]
                sv = jnp.maximum(iv, 0)
                g16 = plsc.load_gather(cent_ref, [sv])
                ob[pl.ds(off, _LANES)] = g16 * mb[pl.ds(off, _LANES)]

            pltpu.async_copy(ob, out_hbm.at[pl.ds(base, _CHUNK)],
                             sem_out[b])

            @pl.when(g < _NCHUNKS // _NBUF - 1)
            def _():
                start_in(c + _NBUF, b)
        return carry

    lax.fori_loop(0, _NCHUNKS // _NBUF, outer, 0)

    for b in range(_NBUF):
        last = base_w + (_NCHUNKS - _NBUF + b) * _CHUNK
        pltpu.make_async_copy(out_bufs[b], out_hbm.at[pl.ds(last, _CHUNK)],
                              sem_out[b]).wait()


def kernel(cent, idx, mask):
    idx_flat = idx.reshape(_N)
    mask_flat = mask.reshape(_N)

    mesh = plsc.VectorSubcoreMesh(core_axis_name="c", subcore_axis_name="s")
    out = pl.kernel(
        _sc_body,
        mesh=mesh,
        compiler_params=pltpu.CompilerParams(needs_layout_passes=False),
        out_type=jax.ShapeDtypeStruct((_N,), jnp.float32),
        scratch_types=[
            pltpu.VMEM((_K,), jnp.float32),
            pltpu.VMEM((_CHUNK,), jnp.int32),
            pltpu.VMEM((_CHUNK,), jnp.int32),
            pltpu.VMEM((_CHUNK,), jnp.float32),
            pltpu.VMEM((_CHUNK,), jnp.float32),
            pltpu.VMEM((_CHUNK,), jnp.float32),
            pltpu.VMEM((_CHUNK,), jnp.float32),
            pltpu.SemaphoreType.DMA,
            pltpu.SemaphoreType.DMA,
            pltpu.SemaphoreType.DMA,
            pltpu.SemaphoreType.DMA,
        ],
    )(cent, idx_flat, mask_flat)
    return out.reshape(_SHAPE)


# speedup vs baseline: 1471.9388x; 2.4124x over previous
import functools

import jax
import jax.numpy as jnp
from jax import lax
from jax.experimental import pallas as pl
from jax.experimental.pallas import tpu as pltpu
from jax.experimental.pallas import tpu_sc as plsc

_K = 8192
_SHAPE = (4096, 4096)
_N = _SHAPE[0] * _SHAPE[1]

_NC = 2
_NS = 16
_NW = _NC * _NS
_LANES = 16

_BR = 8
_BC = 1024
_BLK = _BR * _BC
_NBLK = _N // _BLK
_PER_W = _NBLK // _NW
_CPR = _SHAPE[1] // _BC
_NBUF = 2


def _sc_body(cent_hbm, idx_hbm, mask_hbm, out_hbm,
             cent_ref, idx0, idx1, mask0, mask1, out0, out1,
             sem_in0, sem_in1, sem_out0, sem_out1):
    idx_bufs = (idx0, idx1)
    mask_bufs = (mask0, mask1)
    out_bufs = (out0, out1)
    sem_in = (sem_in0, sem_in1)
    sem_out = (sem_out0, sem_out1)

    wid = lax.axis_index("s") * _NC + lax.axis_index("c")
    base_q = wid * _PER_W

    pltpu.sync_copy(cent_hbm, cent_ref)

    def block_slice(ref, q):
        rb = (q // _CPR) * _BR
        cb = (q % _CPR) * _BC
        return ref.at[pl.ds(rb, _BR), pl.ds(cb, _BC)]

    def start_in(q, b):
        pltpu.async_copy(block_slice(idx_hbm, q), idx_bufs[b], sem_in[b])
        pltpu.async_copy(block_slice(mask_hbm, q), mask_bufs[b], sem_in[b])

    for b in range(_NBUF):
        start_in(base_q + b, b)

    def outer(g, carry):
        for b in range(_NBUF):
            q = base_q + _NBUF * g + b
            ib, mb, ob = idx_bufs[b], mask_bufs[b], out_bufs[b]

            pltpu.make_async_copy(block_slice(idx_hbm, q), ib,
                                  sem_in[b]).wait()
            pltpu.make_async_copy(block_slice(mask_hbm, q), mb,
                                  sem_in[b]).wait()

            @pl.when(g > 0)
            def _():
                pltpu.make_async_copy(ob, block_slice(out_hbm, q - _NBUF),
                                      sem_out[b]).wait()

            for r in range(_BR):
                @plsc.parallel_loop(0, _BC, _LANES, unroll=8)
                def _(off):
                    iv = ib[r, pl.ds(off, _LANES)]
                    sv = jnp.maximum(iv, 0)
                    g16 = plsc.load_gather(cent_ref, [sv])
                    ob[r, pl.ds(off, _LANES)] = g16 * mb[r, pl.ds(off, _LANES)]

            pltpu.async_copy(ob, block_slice(out_hbm, q), sem_out[b])

            @pl.when(g < _PER_W // _NBUF - 1)
            def _():
                start_in(q + _NBUF, b)
        return carry

    lax.fori_loop(0, _PER_W // _NBUF, outer, 0)

    for b in range(_NBUF):
        last = base_q + _PER_W - _NBUF + b
        pltpu.make_async_copy(out_bufs[b], block_slice(out_hbm, last),
                              sem_out[b]).wait()


def kernel(cent, idx, mask):
    mesh = plsc.VectorSubcoreMesh(core_axis_name="c", subcore_axis_name="s")
    out = pl.kernel(
        _sc_body,
        mesh=mesh,
        compiler_params=pltpu.CompilerParams(
            needs_layout_passes=False,
            use_tc_tiling_on_sc=True,
        ),
        out_type=jax.ShapeDtypeStruct(_SHAPE, jnp.float32),
        scratch_types=[
            pltpu.VMEM((_K,), jnp.float32),
            pltpu.VMEM((_BR, _BC), jnp.int32),
            pltpu.VMEM((_BR, _BC), jnp.int32),
            pltpu.VMEM((_BR, _BC), jnp.float32),
            pltpu.VMEM((_BR, _BC), jnp.float32),
            pltpu.VMEM((_BR, _BC), jnp.float32),
            pltpu.VMEM((_BR, _BC), jnp.float32),
            pltpu.SemaphoreType.DMA,
            pltpu.SemaphoreType.DMA,
            pltpu.SemaphoreType.DMA,
            pltpu.SemaphoreType.DMA,
        ],
    )(cent, idx, mask)
    return out


# speedup vs baseline: 1568.1940x; 1.0654x over previous
import functools

import jax
import jax.numpy as jnp
from jax import lax
from jax.experimental import pallas as pl
from jax.experimental.pallas import tpu as pltpu
from jax.experimental.pallas import tpu_sc as plsc

_K = 8192
_SHAPE = (4096, 4096)
_N = _SHAPE[0] * _SHAPE[1]

_NC = 2
_NS = 16
_NW = _NC * _NS
_LANES = 16

_BR = 8
_BC = 2048
_BLK = _BR * _BC
_NBLK = _N // _BLK
_PER_W = _NBLK // _NW
_CPR = _SHAPE[1] // _BC
_NBUF = 2


def _sc_body(cent_hbm, idx_hbm, mask_hbm, out_hbm,
             cent_ref, idx0, idx1, mask0, mask1, out0, out1,
             sem_in0, sem_in1, sem_out0, sem_out1):
    idx_bufs = (idx0, idx1)
    mask_bufs = (mask0, mask1)
    out_bufs = (out0, out1)
    sem_in = (sem_in0, sem_in1)
    sem_out = (sem_out0, sem_out1)

    wid = lax.axis_index("s") * _NC + lax.axis_index("c")
    base_q = wid * _PER_W

    pltpu.sync_copy(cent_hbm, cent_ref)

    def block_slice(ref, q):
        rb = (q // _CPR) * _BR
        cb = (q % _CPR) * _BC
        return ref.at[pl.ds(rb, _BR), pl.ds(cb, _BC)]

    def start_in(q, b):
        pltpu.async_copy(block_slice(idx_hbm, q), idx_bufs[b], sem_in[b])
        pltpu.async_copy(block_slice(mask_hbm, q), mask_bufs[b], sem_in[b])

    for b in range(_NBUF):
        start_in(base_q + b, b)

    def outer(g, carry):
        for b in range(_NBUF):
            q = base_q + _NBUF * g + b
            ib, mb, ob = idx_bufs[b], mask_bufs[b], out_bufs[b]

            pltpu.make_async_copy(block_slice(idx_hbm, q), ib,
                                  sem_in[b]).wait()
            pltpu.make_async_copy(block_slice(mask_hbm, q), mb,
                                  sem_in[b]).wait()

            @pl.when(g > 0)
            def _():
                pltpu.make_async_copy(ob, block_slice(out_hbm, q - _NBUF),
                                      sem_out[b]).wait()

            for r in range(_BR):
                @plsc.parallel_loop(0, _BC, _LANES, unroll=16)
                def _(off):
                    iv = ib[r, pl.ds(off, _LANES)]
                    sv = jnp.maximum(iv, 0)
                    g16 = plsc.load_gather(cent_ref, [sv])
                    ob[r, pl.ds(off, _LANES)] = g16 * mb[r, pl.ds(off, _LANES)]

            pltpu.async_copy(ob, block_slice(out_hbm, q), sem_out[b])

            @pl.when(g < _PER_W // _NBUF - 1)
            def _():
                start_in(q + _NBUF, b)
        return carry

    lax.fori_loop(0, _PER_W // _NBUF, outer, 0)

    for b in range(_NBUF):
        last = base_q + _PER_W - _NBUF + b
        pltpu.make_async_copy(out_bufs[b], block_slice(out_hbm, last),
                              sem_out[b]).wait()


def kernel(cent, idx, mask):
    mesh = plsc.VectorSubcoreMesh(core_axis_name="c", subcore_axis_name="s")
    out = pl.kernel(
        _sc_body,
        mesh=mesh,
        compiler_params=pltpu.CompilerParams(
            needs_layout_passes=False,
            use_tc_tiling_on_sc=True,
        ),
        out_type=jax.ShapeDtypeStruct(_SHAPE, jnp.float32),
        scratch_types=[
            pltpu.VMEM((_K,), jnp.float32),
            pltpu.VMEM((_BR, _BC), jnp.int32),
            pltpu.VMEM((_BR, _BC), jnp.int32),
            pltpu.VMEM((_BR, _BC), jnp.float32),
            pltpu.VMEM((_BR, _BC), jnp.float32),
            pltpu.VMEM((_BR, _BC), jnp.float32),
            pltpu.VMEM((_BR, _BC), jnp.float32),
            pltpu.SemaphoreType.DMA,
            pltpu.SemaphoreType.DMA,
            pltpu.SemaphoreType.DMA,
            pltpu.SemaphoreType.DMA,
        ],
    )(cent, idx, mask)
    return out


# speedup vs baseline: 1665.9127x; 1.0623x over previous
import functools

import jax
import jax.numpy as jnp
from jax import lax
from jax.experimental import pallas as pl
from jax.experimental.pallas import tpu as pltpu
from jax.experimental.pallas import tpu_sc as plsc

_K = 8192
_SHAPE = (4096, 4096)
_N = _SHAPE[0] * _SHAPE[1]

_NC = 2
_NS = 16
_NW = _NC * _NS
_LANES = 16

_BR = 8
_BC = 2048
_BLK = _BR * _BC
_NBLK = _N // _BLK
_PER_W = _NBLK // _NW
_CPR = _SHAPE[1] // _BC
_NBUF = 2


def _sc_body(cent_hbm, idx_hbm, mask_hbm, out_hbm,
             cent_ref, idx0, idx1, mask0, mask1, out0, out1,
             sem_in0, sem_in1, sem_out0, sem_out1):
    idx_bufs = (idx0, idx1)
    mask_bufs = (mask0, mask1)
    out_bufs = (out0, out1)
    sem_in = (sem_in0, sem_in1)
    sem_out = (sem_out0, sem_out1)

    wid = lax.axis_index("s") * _NC + lax.axis_index("c")
    base_q = wid * _PER_W

    pltpu.sync_copy(cent_hbm, cent_ref)

    def block_slice(ref, q):
        rb = (q // _CPR) * _BR
        cb = (q % _CPR) * _BC
        return ref.at[pl.ds(rb, _BR), pl.ds(cb, _BC)]

    def start_in(q, b):
        pltpu.async_copy(block_slice(idx_hbm, q), idx_bufs[b], sem_in[b])
        pltpu.async_copy(block_slice(mask_hbm, q), mask_bufs[b], sem_in[b])

    for b in range(_NBUF):
        start_in(base_q + b, b)

    def outer(g, carry):
        for b in range(_NBUF):
            q = base_q + _NBUF * g + b
            ib, mb, ob = idx_bufs[b], mask_bufs[b], out_bufs[b]

            pltpu.make_async_copy(block_slice(idx_hbm, q), ib,
                                  sem_in[b]).wait()
            pltpu.make_async_copy(block_slice(mask_hbm, q), mb,
                                  sem_in[b]).wait()

            @pl.when(g > 0)
            def _():
                pltpu.make_async_copy(ob, block_slice(out_hbm, q - _NBUF),
                                      sem_out[b]).wait()

            @plsc.parallel_loop(0, _BLK // _LANES, 1, unroll=16)
            def _(qq):
                r = qq >> 7
                off = (qq & 127) * _LANES
                iv = ib[r, pl.ds(off, _LANES)]
                sv = jnp.maximum(iv, 0)
                g16 = plsc.load_gather(cent_ref, [sv])
                ob[r, pl.ds(off, _LANES)] = g16 * mb[r, pl.ds(off, _LANES)]

            pltpu.async_copy(ob, block_slice(out_hbm, q), sem_out[b])

            @pl.when(g < _PER_W // _NBUF - 1)
            def _():
                start_in(q + _NBUF, b)
        return carry

    lax.fori_loop(0, _PER_W // _NBUF, outer, 0)

    for b in range(_NBUF):
        last = base_q + _PER_W - _NBUF + b
        pltpu.make_async_copy(out_bufs[b], block_slice(out_hbm, last),
                              sem_out[b]).wait()


def kernel(cent, idx, mask):
    mesh = plsc.VectorSubcoreMesh(core_axis_name="c", subcore_axis_name="s")
    out = pl.kernel(
        _sc_body,
        mesh=mesh,
        compiler_params=pltpu.CompilerParams(
            needs_layout_passes=False,
            use_tc_tiling_on_sc=True,
        ),
        out_type=jax.ShapeDtypeStruct(_SHAPE, jnp.float32),
        scratch_types=[
            pltpu.VMEM((_K,), jnp.float32),
            pltpu.VMEM((_BR, _BC), jnp.int32),
            pltpu.VMEM((_BR, _BC), jnp.int32),
            pltpu.VMEM((_BR, _BC), jnp.float32),
            pltpu.VMEM((_BR, _BC), jnp.float32),
            pltpu.VMEM((_BR, _BC), jnp.float32),
            pltpu.VMEM((_BR, _BC), jnp.float32),
            pltpu.SemaphoreType.DMA,
            pltpu.SemaphoreType.DMA,
            pltpu.SemaphoreType.DMA,
            pltpu.SemaphoreType.DMA,
        ],
    )(cent, idx, mask)
    return out
